# baseline (speedup 1.0000x reference)
import functools

import jax
import jax.numpy as jnp
from jax import lax
from jax.experimental import pallas as pl
from jax.experimental.pallas import tpu as pltpu
from jax.experimental.pallas import tpu_sc as plsc

VOCAB = 1000000
EMBED_DIM = 64
NUM_CLASS = 10
BATCH = 4096
HIST = 50

NC = 2
NS = 16
NW = NC * NS
BAGS_PER_W = BATCH // NW
CHUNK_BAGS = 2
CHUNK_IDX = CHUNK_BAGS * HIST
NCHUNK = BAGS_PER_W // CHUNK_BAGS
LANES = 16
NVREG = EMBED_DIM // LANES


def _sc_body(text_hbm, table_hbm, out_hbm, idx_v, rows_v, means_v, sem):
    wid = lax.axis_index("s") * NC + lax.axis_index("c")

    pltpu.sync_copy(text_hbm.at[wid], idx_v)

    inv_l = jnp.float32(1.0 / HIST)

    def chunk_body(c, _):
        pltpu.async_copy(table_hbm.at[idx_v.at[c]], rows_v, sem).wait()
        for b in range(CHUNK_BAGS):
            base = b * HIST

            def racc(r, accs):
                return tuple(
                    accs[k] + rows_v[base + r, pl.ds(k * LANES, LANES)]
                    for k in range(NVREG)
                )

            zero = jnp.zeros((LANES,), jnp.float32)
            accs = lax.fori_loop(0, HIST, racc, (zero,) * NVREG)
            slot = c * CHUNK_BAGS + b
            for k in range(NVREG):
                means_v[slot, pl.ds(k * LANES, LANES)] = accs[k] * inv_l
        return _

    lax.fori_loop(0, NCHUNK, chunk_body, None)

    pltpu.sync_copy(means_v, out_hbm.at[pl.ds(wid * BAGS_PER_W, BAGS_PER_W)])


@jax.jit
def _sc_embed_mean(text3, table):
    mesh = plsc.VectorSubcoreMesh(core_axis_name="c", subcore_axis_name="s")
    return pl.kernel(
        _sc_body,
        out_type=jax.ShapeDtypeStruct((BATCH, EMBED_DIM), jnp.float32),
        mesh=mesh,
        scratch_types=[
            pltpu.VMEM((NCHUNK, CHUNK_IDX), jnp.int32),
            pltpu.VMEM((CHUNK_IDX, EMBED_DIM), jnp.float32),
            pltpu.VMEM((BAGS_PER_W, EMBED_DIM), jnp.float32),
            pltpu.SemaphoreType.DMA,
        ],
        compiler_params=pltpu.CompilerParams(use_tc_tiling_on_sc=False),
    )(text3, table)


def _tc_linear_body(x_ref, wt_ref, b_ref, o_ref):
    o_ref[...] = (
        jnp.dot(x_ref[...], wt_ref[...], preferred_element_type=jnp.float32)
        + b_ref[...]
    )


@jax.jit
def _tc_linear(x, wt, b2d):
    return pl.pallas_call(
        _tc_linear_body,
        out_shape=jax.ShapeDtypeStruct((BATCH, NUM_CLASS), jnp.float32),
    )(x, wt, b2d)


def kernel(text, emb_table, fc_w, fc_b):
    text3 = text.astype(jnp.int32).reshape(NW, NCHUNK, CHUNK_IDX)
    means = _sc_embed_mean(text3, emb_table)
    return _tc_linear(means, fc_w.T, fc_b.reshape(1, NUM_CLASS))


# speedup vs baseline: 1.8021x; 1.8021x over previous
import functools

import jax
import jax.numpy as jnp
from jax import lax
from jax.experimental import pallas as pl
from jax.experimental.pallas import tpu as pltpu
from jax.experimental.pallas import tpu_sc as plsc

VOCAB = 1000000
EMBED_DIM = 64
NUM_CLASS = 10
BATCH = 4096
HIST = 50

NC = 2
NS = 16
NW = NC * NS
BAGS_PER_W = BATCH // NW
CHUNK_BAGS = 2
CHUNK_IDX = CHUNK_BAGS * HIST
NCHUNK = BAGS_PER_W // CHUNK_BAGS
LANES = 16

VB = 8192
NBLK = (VOCAB + VB - 1) // VB
SROWS = VOCAB // 8


def _project_body(tt_ref, w_ref, o_ref):
    st = jax.lax.dot_general(
        tt_ref[...], w_ref[...],
        dimension_numbers=(((0,), (1,)), ((), ())),
        preferred_element_type=jnp.float32,
    )
    st3 = st.reshape(VB // 8, 8, LANES)
    for s in range(8):
        o_ref[:, s * LANES:(s + 1) * LANES] = st3[:, s, :]


@jax.jit
def _tc_project(table_t, w16):
    return pl.pallas_call(
        _project_body,
        grid=(NBLK,),
        in_specs=[
            pl.BlockSpec((EMBED_DIM, VB), lambda i: (0, i)),
            pl.BlockSpec((LANES, EMBED_DIM), lambda i: (0, 0)),
        ],
        out_specs=pl.BlockSpec((VB // 8, 128), lambda i: (i, 0)),
        out_shape=jax.ShapeDtypeStruct((SROWS, 128), jnp.float32),
    )(table_t, w16)


def _sc_body(text_hbm, s_hbm, b_hbm, out_hbm, idx_v, rows_v, means_v, b_v, sem):
    wid = lax.axis_index("s") * NC + lax.axis_index("c")

    pltpu.sync_copy(text_hbm.at[wid], idx_v)
    pltpu.sync_copy(b_hbm, b_v)
    bias = b_v[...]

    def chunk_body(c, _):
        pltpu.async_copy(s_hbm.at[idx_v.at[c]], rows_v, sem).wait()
        for b in range(CHUNK_BAGS):
            base = b * HIST

            def racc(r, acc):
                return acc + rows_v[base + r, :]

            acc = lax.fori_loop(0, HIST, racc, jnp.zeros((LANES,), jnp.float32))
            means_v[c * CHUNK_BAGS + b, :] = acc + bias
        return _

    lax.fori_loop(0, NCHUNK, chunk_body, None)

    pltpu.sync_copy(means_v, out_hbm.at[pl.ds(wid * BAGS_PER_W, BAGS_PER_W)])


@jax.jit
def _sc_gather_mean(text3, s_lin, b16):
    mesh = plsc.VectorSubcoreMesh(core_axis_name="c", subcore_axis_name="s")
    return pl.kernel(
        _sc_body,
        out_type=jax.ShapeDtypeStruct((BATCH, LANES), jnp.float32),
        mesh=mesh,
        scratch_types=[
            pltpu.VMEM((NCHUNK, CHUNK_IDX), jnp.int32),
            pltpu.VMEM((CHUNK_IDX, LANES), jnp.float32),
            pltpu.VMEM((BAGS_PER_W, LANES), jnp.float32),
            pltpu.VMEM((LANES,), jnp.float32),
            pltpu.SemaphoreType.DMA,
        ],
        compiler_params=pltpu.CompilerParams(use_tc_tiling_on_sc=False),
    )(text3, s_lin, b16)


def kernel(text, emb_table, fc_w, fc_b):
    text3 = text.astype(jnp.int32).reshape(NW, NCHUNK, CHUNK_IDX)
    table_t = emb_table.T
    w16 = jnp.pad(fc_w, ((0, LANES - NUM_CLASS), (0, 0))) * (1.0 / HIST)
    b16 = jnp.pad(fc_b, (0, LANES - NUM_CLASS))
    s = _tc_project(table_t, w16)
    s_lin = s.reshape(VOCAB, LANES)
    means = _sc_gather_mean(text3, s_lin, b16)
    return means[:, :NUM_CLASS]


# speedup vs baseline: 1.9572x; 1.0861x over previous
import functools

import jax
import jax.numpy as jnp
from jax import lax
from jax.experimental import pallas as pl
from jax.experimental.pallas import tpu as pltpu
from jax.experimental.pallas import tpu_sc as plsc

VOCAB = 1000000
EMBED_DIM = 64
NUM_CLASS = 10
BATCH = 4096
HIST = 50

NC = 2
NS = 16
NW = NC * NS
BAGS_PER_W = BATCH // NW
CHUNK_BAGS = 2
CHUNK_IDX = CHUNK_BAGS * HIST
NCHUNK = BAGS_PER_W // CHUNK_BAGS
LANES = 16

VB = 8192
NBLK = (VOCAB + VB - 1) // VB
SROWS = VOCAB // 8


def _project_body(tt_ref, w_ref, o_ref, st_ref):
    st_ref[...] = jnp.dot(
        tt_ref[...].T, w_ref[...], preferred_element_type=jnp.float32
    )
    for s in range(8):
        o_ref[:, s * LANES:(s + 1) * LANES] = st_ref[pl.ds(s, VB // 8, 8), :]


@jax.jit
def _tc_project(table_t, w16):
    return pl.pallas_call(
        _project_body,
        grid=(NBLK,),
        in_specs=[
            pl.BlockSpec((EMBED_DIM, VB), lambda i: (0, i)),
            pl.BlockSpec((EMBED_DIM, LANES), lambda i: (0, 0)),
        ],
        out_specs=pl.BlockSpec((VB // 8, 128), lambda i: (i, 0)),
        out_shape=jax.ShapeDtypeStruct((SROWS, 128), jnp.float32),
        scratch_shapes=[pltpu.VMEM((VB, LANES), jnp.float32)],
        compiler_params=pltpu.CompilerParams(fuse_transposed_lhs_in_matmul=True),
    )(table_t, w16)


NBUF = 4


def _sc_body(text_hbm, s_hbm, b_hbm, out_hbm, idx_v, r0, r1, r2, r3, means_v,
             b_v, sems):
    wid = lax.axis_index("s") * NC + lax.axis_index("c")
    bufs = (r0, r1, r2, r3)

    pltpu.sync_copy(text_hbm.at[wid], idx_v)
    pltpu.sync_copy(b_hbm, b_v)
    bias = b_v[...]

    def gather(c, b):
        return pltpu.make_async_copy(s_hbm.at[idx_v.at[c]], bufs[b], sems.at[b])

    for b in range(NBUF):
        gather(b, b).start()

    def round_body(c4, _):
        for b in range(NBUF):
            c = c4 * NBUF + b
            gather(c, b).wait()
            for g in range(CHUNK_BAGS):
                base = g * HIST

                def racc(r, acc):
                    return acc + bufs[b][base + r, :]

                acc = lax.fori_loop(0, HIST, racc,
                                    jnp.zeros((LANES,), jnp.float32))
                means_v[c * CHUNK_BAGS + g, :] = acc + bias

            @pl.when(c + NBUF < NCHUNK)
            def _start_next():
                gather(c + NBUF, b).start()
        return _

    lax.fori_loop(0, NCHUNK // NBUF, round_body, None)

    pltpu.sync_copy(means_v, out_hbm.at[pl.ds(wid * BAGS_PER_W, BAGS_PER_W)])


@jax.jit
def _sc_gather_mean(text3, s_lin, b16):
    mesh = plsc.VectorSubcoreMesh(core_axis_name="c", subcore_axis_name="s")
    return pl.kernel(
        _sc_body,
        out_type=jax.ShapeDtypeStruct((BATCH, LANES), jnp.float32),
        mesh=mesh,
        scratch_types=[
            pltpu.VMEM((NCHUNK, CHUNK_IDX), jnp.int32),
            pltpu.VMEM((CHUNK_IDX, LANES), jnp.float32),
            pltpu.VMEM((CHUNK_IDX, LANES), jnp.float32),
            pltpu.VMEM((CHUNK_IDX, LANES), jnp.float32),
            pltpu.VMEM((CHUNK_IDX, LANES), jnp.float32),
            pltpu.VMEM((BAGS_PER_W, LANES), jnp.float32),
            pltpu.VMEM((LANES,), jnp.float32),
            pltpu.SemaphoreType.DMA((NBUF,)),
        ],
        compiler_params=pltpu.CompilerParams(use_tc_tiling_on_sc=False),
    )(text3, s_lin, b16)


def kernel(text, emb_table, fc_w, fc_b):
    text3 = text.astype(jnp.int32).reshape(NW, NCHUNK, CHUNK_IDX)
    table_t = emb_table.T
    w16 = jnp.pad(fc_w.T, ((0, 0), (0, LANES - NUM_CLASS))) * (1.0 / HIST)
    b16 = jnp.pad(fc_b, (0, LANES - NUM_CLASS))
    s = _tc_project(table_t, w16)
    s_lin = s.reshape(VOCAB, LANES)
    means = _sc_gather_mean(text3, s_lin, b16)
    return means[:, :NUM_CLASS]


# speedup vs baseline: 2.0090x; 1.0265x over previous
import functools

import jax
import jax.numpy as jnp
from jax import lax
from jax.experimental import pallas as pl
from jax.experimental.pallas import tpu as pltpu
from jax.experimental.pallas import tpu_sc as plsc

VOCAB = 1000000
EMBED_DIM = 64
NUM_CLASS = 10
BATCH = 4096
HIST = 50

NC = 2
NS = 16
NW = NC * NS
BAGS_PER_W = BATCH // NW
CHUNK_BAGS = 2
CHUNK_IDX = CHUNK_BAGS * HIST
NCHUNK = BAGS_PER_W // CHUNK_BAGS
LANES = 16

VB = 16384
NBLK = (VOCAB + VB - 1) // VB
SROWS = VOCAB // 8


def _project_body(tt_ref, w_ref, o_ref, st_ref):
    st_ref[...] = jnp.dot(
        tt_ref[...].T, w_ref[...], preferred_element_type=jnp.float32
    )
    for s in range(8):
        o_ref[:, s * LANES:(s + 1) * LANES] = st_ref[pl.ds(s, VB // 8, 8), :]


@jax.jit
def _tc_project(table_t, w16):
    return pl.pallas_call(
        _project_body,
        grid=(NBLK,),
        in_specs=[
            pl.BlockSpec((EMBED_DIM, VB), lambda i: (0, i)),
            pl.BlockSpec((EMBED_DIM, LANES), lambda i: (0, 0)),
        ],
        out_specs=pl.BlockSpec((VB // 8, 128), lambda i: (i, 0)),
        out_shape=jax.ShapeDtypeStruct((SROWS, 128), jnp.float32),
        scratch_shapes=[pltpu.VMEM((VB, LANES), jnp.float32)],
        compiler_params=pltpu.CompilerParams(fuse_transposed_lhs_in_matmul=True),
    )(table_t, w16)


NBUF = 4


def _sc_body(text_hbm, s_hbm, b_hbm, out_hbm, idx_v, r0, r1, r2, r3, means_v,
             b_v, sems):
    wid = lax.axis_index("s") * NC + lax.axis_index("c")
    bufs = (r0, r1, r2, r3)

    pltpu.sync_copy(text_hbm.at[wid], idx_v)
    pltpu.sync_copy(b_hbm, b_v)
    bias = b_v[...]

    def gather(c, b):
        return pltpu.make_async_copy(s_hbm.at[idx_v.at[c]], bufs[b], sems.at[b])

    for b in range(NBUF):
        gather(b, b).start()

    def round_body(c4, _):
        for b in range(NBUF):
            c = c4 * NBUF + b
            gather(c, b).wait()
            for g in range(CHUNK_BAGS):
                base = g * HIST

                def racc(r, acc):
                    return acc + bufs[b][base + r, :]

                acc = lax.fori_loop(0, HIST, racc,
                                    jnp.zeros((LANES,), jnp.float32))
                means_v[c * CHUNK_BAGS + g, :] = acc + bias

            @pl.when(c + NBUF < NCHUNK)
            def _start_next():
                gather(c + NBUF, b).start()
        return _

    lax.fori_loop(0, NCHUNK // NBUF, round_body, None)

    pltpu.sync_copy(means_v, out_hbm.at[pl.ds(wid * BAGS_PER_W, BAGS_PER_W)])


@jax.jit
def _sc_gather_mean(text3, s_lin, b16):
    mesh = plsc.VectorSubcoreMesh(core_axis_name="c", subcore_axis_name="s")
    return pl.kernel(
        _sc_body,
        out_type=jax.ShapeDtypeStruct((BATCH, LANES), jnp.float32),
        mesh=mesh,
        scratch_types=[
            pltpu.VMEM((NCHUNK, CHUNK_IDX), jnp.int32),
            pltpu.VMEM((CHUNK_IDX, LANES), jnp.float32),
            pltpu.VMEM((CHUNK_IDX, LANES), jnp.float32),
            pltpu.VMEM((CHUNK_IDX, LANES), jnp.float32),
            pltpu.VMEM((CHUNK_IDX, LANES), jnp.float32),
            pltpu.VMEM((BAGS_PER_W, LANES), jnp.float32),
            pltpu.VMEM((LANES,), jnp.float32),
            pltpu.SemaphoreType.DMA((NBUF,)),
        ],
        compiler_params=pltpu.CompilerParams(use_tc_tiling_on_sc=False),
    )(text3, s_lin, b16)


def kernel(text, emb_table, fc_w, fc_b):
    text3 = text.astype(jnp.int32).reshape(NW, NCHUNK, CHUNK_IDX)
    table_t = emb_table.T
    w16 = jnp.pad(fc_w.T, ((0, 0), (0, LANES - NUM_CLASS))) * (1.0 / HIST)
    b16 = jnp.pad(fc_b, (0, LANES - NUM_CLASS))
    s = _tc_project(table_t, w16)
    s_lin = s.reshape(VOCAB, LANES)
    means = _sc_gather_mean(text3, s_lin, b16)
    return means[:, :NUM_CLASS]


# speedup vs baseline: 2.0176x; 1.0043x over previous
import functools

import jax
import jax.numpy as jnp
from jax import lax
from jax.experimental import pallas as pl
from jax.experimental.pallas import tpu as pltpu
from jax.experimental.pallas import tpu_sc as plsc

VOCAB = 1000000
EMBED_DIM = 64
NUM_CLASS = 10
BATCH = 4096
HIST = 50

NC = 2
NS = 16
NW = NC * NS
BAGS_PER_W = BATCH // NW
CHUNK_BAGS = 2
CHUNK_IDX = CHUNK_BAGS * HIST
NCHUNK = BAGS_PER_W // CHUNK_BAGS
LANES = 16

VB = 32768
NBLK = (VOCAB + VB - 1) // VB
SROWS = VOCAB // 8


def _project_body(tt_ref, w_ref, o_ref, st_ref):
    st_ref[...] = jnp.dot(
        tt_ref[...].T, w_ref[...], preferred_element_type=jnp.float32
    )
    for s in range(8):
        o_ref[:, s * LANES:(s + 1) * LANES] = st_ref[pl.ds(s, VB // 8, 8), :]


@jax.jit
def _tc_project(table_t, w16):
    return pl.pallas_call(
        _project_body,
        grid=(NBLK,),
        in_specs=[
            pl.BlockSpec((EMBED_DIM, VB), lambda i: (0, i)),
            pl.BlockSpec((EMBED_DIM, LANES), lambda i: (0, 0)),
        ],
        out_specs=pl.BlockSpec((VB // 8, 128), lambda i: (i, 0)),
        out_shape=jax.ShapeDtypeStruct((SROWS, 128), jnp.float32),
        scratch_shapes=[pltpu.VMEM((VB, LANES), jnp.float32)],
        compiler_params=pltpu.CompilerParams(fuse_transposed_lhs_in_matmul=True),
    )(table_t, w16)


NBUF = 4


def _sc_body(text_hbm, s_hbm, b_hbm, out_hbm, idx_v, r0, r1, r2, r3, means_v,
             b_v, sems):
    wid = lax.axis_index("s") * NC + lax.axis_index("c")
    bufs = (r0, r1, r2, r3)

    pltpu.sync_copy(text_hbm.at[wid], idx_v)
    pltpu.sync_copy(b_hbm, b_v)
    bias = b_v[...]

    def gather(c, b):
        return pltpu.make_async_copy(s_hbm.at[idx_v.at[c]], bufs[b], sems.at[b])

    for b in range(NBUF):
        gather(b, b).start()

    def round_body(c4, _):
        for b in range(NBUF):
            c = c4 * NBUF + b
            gather(c, b).wait()
            for g in range(CHUNK_BAGS):
                base = g * HIST

                def racc(r, acc):
                    return acc + bufs[b][base + r, :]

                acc = lax.fori_loop(0, HIST, racc,
                                    jnp.zeros((LANES,), jnp.float32))
                means_v[c * CHUNK_BAGS + g, :] = acc + bias

            @pl.when(c + NBUF < NCHUNK)
            def _start_next():
                gather(c + NBUF, b).start()
        return _

    lax.fori_loop(0, NCHUNK // NBUF, round_body, None)

    pltpu.sync_copy(means_v, out_hbm.at[pl.ds(wid * BAGS_PER_W, BAGS_PER_W)])


@jax.jit
def _sc_gather_mean(text3, s_lin, b16):
    mesh = plsc.VectorSubcoreMesh(core_axis_name="c", subcore_axis_name="s")
    return pl.kernel(
        _sc_body,
        out_type=jax.ShapeDtypeStruct((BATCH, LANES), jnp.float32),
        mesh=mesh,
        scratch_types=[
            pltpu.VMEM((NCHUNK, CHUNK_IDX), jnp.int32),
            pltpu.VMEM((CHUNK_IDX, LANES), jnp.float32),
            pltpu.VMEM((CHUNK_IDX, LANES), jnp.float32),
            pltpu.VMEM((CHUNK_IDX, LANES), jnp.float32),
            pltpu.VMEM((CHUNK_IDX, LANES), jnp.float32),
            pltpu.VMEM((BAGS_PER_W, LANES), jnp.float32),
            pltpu.VMEM((LANES,), jnp.float32),
            pltpu.SemaphoreType.DMA((NBUF,)),
        ],
        compiler_params=pltpu.CompilerParams(use_tc_tiling_on_sc=False),
    )(text3, s_lin, b16)


def kernel(text, emb_table, fc_w, fc_b):
    text3 = text.astype(jnp.int32).reshape(NW, NCHUNK, CHUNK_IDX)
    table_t = emb_table.T
    w16 = jnp.pad(fc_w.T, ((0, 0), (0, LANES - NUM_CLASS))) * (1.0 / HIST)
    b16 = jnp.pad(fc_b, (0, LANES - NUM_CLASS))
    s = _tc_project(table_t, w16)
    s_lin = s.reshape(VOCAB, LANES)
    means = _sc_gather_mean(text3, s_lin, b16)
    return means[:, :NUM_CLASS]
